# baseline (speedup 1.0000x reference)
import functools

import jax
import jax.numpy as jnp
from jax import lax
from jax.experimental import pallas as pl
from jax.experimental.pallas import tpu as pltpu
from jax.experimental.pallas import tpu_sc as plsc

N_NODES = 10000
N_EDGES = 320000
D = 128

NC = 2
NS = 16
NW = NC * NS
EPW = N_EDGES // NW
CH = 80
KC = EPW // CH
NBG = 2
IS = 2 * NBG
KCE = -(-KC // IS)
CD = 80
KD = EPW // CD
RPT = 624
TAIL0 = RPT * NS
TAILN = N_NODES - TAIL0
NP = 10240
RPD = NP // NS

RB = 2000
GRID = N_NODES // RB


def _sc_mesh():
    return plsc.VectorSubcoreMesh(core_axis_name="c", subcore_axis_name="s")


def _deg_body(dst_hbm, out0_hbm, out1_hbm, dst_v, ones_v, stage_v, acc):
    c = lax.axis_index("c")
    s = lax.axis_index("s")
    w = c * NS + s
    for i in range(CD // 16):
        ones_v[pl.ds(i * 16, 16)] = jnp.ones((16,), jnp.float32)
    for i in range(RPD // 16):
        stage_v[pl.ds(i * 16, 16)] = jnp.zeros((16,), jnp.float32)
    pltpu.sync_copy(stage_v, acc.at[pl.ds(s * RPD, RPD)])
    pltpu.sync_copy(dst_hbm.at[w], dst_v)
    plsc.subcore_barrier()

    def body(j, carry):
        pltpu.sync_copy(ones_v, acc.at[dst_v.at[j]], add=True)
        return carry

    lax.fori_loop(0, KD, body, 0)
    plsc.subcore_barrier()
    pltpu.sync_copy(acc.at[pl.ds(s * RPD, RPD)], stage_v)

    @pl.when(c == 0)
    def _():
        pltpu.sync_copy(stage_v, out0_hbm.at[pl.ds(s * RPD, RPD)])

    @pl.when(c == 1)
    def _():
        pltpu.sync_copy(stage_v, out1_hbm.at[pl.ds(s * RPD, RPD)])


def _deg_call(dst3):
    k = functools.partial(
        pl.kernel,
        out_type=[
            jax.ShapeDtypeStruct((NP,), jnp.float32),
            jax.ShapeDtypeStruct((NP,), jnp.float32),
        ],
        mesh=_sc_mesh(),
        scratch_types=[
            pltpu.VMEM((KD, CD), jnp.int32),
            pltpu.VMEM((CD,), jnp.float32),
            pltpu.VMEM((RPD,), jnp.float32),
            pltpu.VMEM_SHARED((NP,), jnp.float32),
        ],
    )(_deg_body)
    return k(dst3)


def _agg_body(sh_hbm, src_hbm, dst_hbm, out_hbm, sidx, didx, gbuf, acc,
              semi, semg):
    c = lax.axis_index("c")
    s = lax.axis_index("s")
    w = c * NS + s
    r0 = s * RPT
    pltpu.sync_copy(sh_hbm.at[pl.ds(r0, RPT)], acc.at[pl.ds(r0, RPT)])

    @pl.when(s == NS - 1)
    def _():
        pltpu.sync_copy(sh_hbm.at[pl.ds(TAIL0, TAILN)],
                        acc.at[pl.ds(TAIL0, TAILN)])

    plsc.subcore_barrier()

    for i in range(IS):
        pltpu.async_copy(src_hbm.at[w].at[i], sidx.at[i], semi.at[i])
        pltpu.async_copy(dst_hbm.at[w].at[i], didx.at[i], semi.at[i])
    for i in range(NBG):
        pltpu.make_async_copy(src_hbm.at[w].at[i], sidx.at[i],
                              semi.at[i]).wait()
        pltpu.make_async_copy(dst_hbm.at[w].at[i], didx.at[i],
                              semi.at[i]).wait()
        pltpu.async_copy(sh_hbm.at[sidx.at[i]], gbuf.at[i], semg.at[i])

    def body(j0, carry):
        for i in range(IS):
            j = j0 * IS + i
            b = i % NBG

            @pl.when(j < KC)
            def _():
                pltpu.make_async_copy(sh_hbm.at[sidx.at[i]], gbuf.at[b],
                                      semg.at[b]).wait()
                pltpu.sync_copy(gbuf.at[b], acc.at[didx.at[i]], add=True)

            @pl.when(j + IS < KC)
            def _():
                pltpu.async_copy(src_hbm.at[w].at[j + IS], sidx.at[i],
                                 semi.at[i])
                pltpu.async_copy(dst_hbm.at[w].at[j + IS], didx.at[i],
                                 semi.at[i])

            i2 = (i + NBG) % IS

            @pl.when(j + NBG < KC)
            def _():
                pltpu.make_async_copy(src_hbm.at[w].at[j + NBG], sidx.at[i2],
                                      semi.at[i2]).wait()
                pltpu.make_async_copy(dst_hbm.at[w].at[j + NBG], didx.at[i2],
                                      semi.at[i2]).wait()
                pltpu.async_copy(sh_hbm.at[sidx.at[i2]], gbuf.at[b],
                                 semg.at[b])
        return carry

    lax.fori_loop(0, KCE, body, 0)
    plsc.subcore_barrier()
    pltpu.sync_copy(acc.at[pl.ds(r0, RPT)], out_hbm.at[c].at[pl.ds(r0, RPT)])

    @pl.when(s == NS - 1)
    def _():
        pltpu.sync_copy(acc.at[pl.ds(TAIL0, TAILN)],
                        out_hbm.at[c].at[pl.ds(TAIL0, TAILN)])


def _agg_call(sh, src3, dst3):
    k = functools.partial(
        pl.kernel,
        out_type=jax.ShapeDtypeStruct((NC, N_NODES, D), jnp.float32),
        mesh=_sc_mesh(),
        scratch_types=[
            pltpu.VMEM((IS, CH), jnp.int32),
            pltpu.VMEM((IS, CH), jnp.int32),
            pltpu.VMEM((NBG, CH, D), jnp.float32),
            pltpu.VMEM_SHARED((N_NODES, D), jnp.float32),
            pltpu.SemaphoreType.DMA((IS,)),
            pltpu.SemaphoreType.DMA((NBG,)),
        ],
    )(_agg_body)
    return k(sh, src3, dst3)


def _pre1_kernel(degT_ref, x_ref, w1_ref, sh_ref, dinv_ref):
    deg = jnp.sum(degT_ref[...], axis=1, keepdims=True) + 1.0
    dinv = lax.rsqrt(deg)
    h = jnp.dot(x_ref[...], w1_ref[...], preferred_element_type=jnp.float32)
    sh_ref[...] = dinv * h
    dinv_ref[...] = dinv


def _pre1(degT, x, W1):
    return pl.pallas_call(
        _pre1_kernel,
        grid=(GRID,),
        in_specs=[
            pl.BlockSpec((RB, NC), lambda i: (i, 0)),
            pl.BlockSpec((RB, D), lambda i: (i, 0)),
            pl.BlockSpec((D, D), lambda i: (0, 0)),
        ],
        out_specs=[
            pl.BlockSpec((RB, D), lambda i: (i, 0)),
            pl.BlockSpec((RB, 1), lambda i: (i, 0)),
        ],
        out_shape=[
            jax.ShapeDtypeStruct((N_NODES, D), jnp.float32),
            jax.ShapeDtypeStruct((N_NODES, 1), jnp.float32),
        ],
    )(degT, x, W1)


def _comb1_kernel(p_ref, sh_ref, dinv_ref, b1_ref, t_ref, stats_ref):
    agg = p_ref[0] + p_ref[1] - sh_ref[...]
    t = dinv_ref[...] * agg + b1_ref[...]
    t_ref[...] = t

    @pl.when(pl.program_id(0) == 0)
    def _():
        stats_ref[...] = jnp.zeros_like(stats_ref)

    stats_ref[0:1, :] += jnp.sum(t, axis=0, keepdims=True)
    stats_ref[1:2, :] += jnp.sum(t * t, axis=0, keepdims=True)


def _comb1(p01, sh1, dinv, b1):
    return pl.pallas_call(
        _comb1_kernel,
        grid=(GRID,),
        in_specs=[
            pl.BlockSpec((NC, RB, D), lambda i: (0, i, 0)),
            pl.BlockSpec((RB, D), lambda i: (i, 0)),
            pl.BlockSpec((RB, 1), lambda i: (i, 0)),
            pl.BlockSpec((1, D), lambda i: (0, 0)),
        ],
        out_specs=[
            pl.BlockSpec((RB, D), lambda i: (i, 0)),
            pl.BlockSpec((8, D), lambda i: (0, 0)),
        ],
        out_shape=[
            jax.ShapeDtypeStruct((N_NODES, D), jnp.float32),
            jax.ShapeDtypeStruct((8, D), jnp.float32),
        ],
    )(p01, sh1, dinv, b1)


def _bn2_kernel(t_ref, stats_ref, g_ref, be_ref, dinv_ref, w2_ref, sh2_ref):
    n = jnp.float32(N_NODES)
    mean = stats_ref[0:1, :] / n
    var = stats_ref[1:2, :] / n - mean * mean
    inv = lax.rsqrt(var + 1e-5)
    h = (t_ref[...] - mean) * inv * g_ref[...] + be_ref[...]
    h = jnp.maximum(h, 0.0)
    sh2_ref[...] = dinv_ref[...] * jnp.dot(
        h, w2_ref[...], preferred_element_type=jnp.float32)


def _bn2(t, stats, gamma1, beta1, dinv, W2):
    return pl.pallas_call(
        _bn2_kernel,
        grid=(GRID,),
        in_specs=[
            pl.BlockSpec((RB, D), lambda i: (i, 0)),
            pl.BlockSpec((8, D), lambda i: (0, 0)),
            pl.BlockSpec((1, D), lambda i: (0, 0)),
            pl.BlockSpec((1, D), lambda i: (0, 0)),
            pl.BlockSpec((RB, 1), lambda i: (i, 0)),
            pl.BlockSpec((D, D), lambda i: (0, 0)),
        ],
        out_specs=pl.BlockSpec((RB, D), lambda i: (i, 0)),
        out_shape=jax.ShapeDtypeStruct((N_NODES, D), jnp.float32),
    )(t, stats, gamma1, beta1, dinv, W2)


def _head_kernel(q_ref, sh2_ref, dinv_ref, b2_ref, wp1_ref, bp1_ref, wp2_ref,
                 bp2_ref, out_ref):
    agg = q_ref[0] + q_ref[1] - sh2_ref[...]
    h = dinv_ref[...] * agg + b2_ref[...]
    p = jnp.tanh(
        jnp.dot(h, wp1_ref[...], preferred_element_type=jnp.float32)
        + bp1_ref[...])
    out_ref[...] = jnp.dot(
        p, wp2_ref[...], preferred_element_type=jnp.float32) + bp2_ref[...]


def _head(q01, sh2, dinv, b2, Wp1, bp1, Wp2, bp2):
    return pl.pallas_call(
        _head_kernel,
        grid=(GRID,),
        in_specs=[
            pl.BlockSpec((NC, RB, D), lambda i: (0, i, 0)),
            pl.BlockSpec((RB, D), lambda i: (i, 0)),
            pl.BlockSpec((RB, 1), lambda i: (i, 0)),
            pl.BlockSpec((1, D), lambda i: (0, 0)),
            pl.BlockSpec((D, D), lambda i: (0, 0)),
            pl.BlockSpec((1, D), lambda i: (0, 0)),
            pl.BlockSpec((D, D), lambda i: (0, 0)),
            pl.BlockSpec((1, D), lambda i: (0, 0)),
        ],
        out_specs=pl.BlockSpec((RB, D), lambda i: (i, 0)),
        out_shape=jax.ShapeDtypeStruct((N_NODES, D), jnp.float32),
    )(q01, sh2, dinv, b2, Wp1, bp1, Wp2, bp2)


def kernel(x, edge_index, W1, b1, gamma1, beta1, W2, b2, Wp1, bp1, Wp2, bp2):
    ei = edge_index.astype(jnp.int32)
    src3 = ei[0].reshape(NW, KC, CH)
    dst3 = ei[1].reshape(NW, KC, CH)
    dst3d = ei[1].reshape(NW, KD, CD)
    b1r = b1.reshape(1, D)
    b2r = b2.reshape(1, D)
    g1r = gamma1.reshape(1, D)
    be1r = beta1.reshape(1, D)
    bp1r = bp1.reshape(1, D)
    bp2r = bp2.reshape(1, D)

    deg0, deg1 = _deg_call(dst3d)
    degT = jnp.stack([deg0, deg1], axis=1)
    sh1, dinv = _pre1(degT, x, W1)
    p01 = _agg_call(sh1, src3, dst3)
    t, stats = _comb1(p01, sh1, dinv, b1r)
    sh2 = _bn2(t, stats, g1r, be1r, dinv, W2)
    q01 = _agg_call(sh2, src3, dst3)
    return _head(q01, sh2, dinv, b2r, Wp1, bp1r, Wp2, bp2r)


# speedup vs baseline: 1.1266x; 1.1266x over previous
import functools

import jax
import jax.numpy as jnp
from jax import lax
from jax.experimental import pallas as pl
from jax.experimental.pallas import tpu as pltpu
from jax.experimental.pallas import tpu_sc as plsc

N_NODES = 10000
N_EDGES = 320000
D = 128

NC = 2
NS = 16
NW = NC * NS
EPW = N_EDGES // NW
CH = 40
KC = EPW // CH
NBG = 4
IS = 2 * NBG
KCE = -(-KC // IS)
CD = 80
KD = EPW // CD
RPT = 624
TAIL0 = RPT * NS
TAILN = N_NODES - TAIL0
NP = 10240
RPD = NP // NS

RB = 2000
GRID = N_NODES // RB


def _sc_mesh():
    return plsc.VectorSubcoreMesh(core_axis_name="c", subcore_axis_name="s")


def _deg_body(dst_hbm, out0_hbm, out1_hbm, dst_v, ones_v, stage_v, acc):
    c = lax.axis_index("c")
    s = lax.axis_index("s")
    w = c * NS + s
    for i in range(CD // 16):
        ones_v[pl.ds(i * 16, 16)] = jnp.ones((16,), jnp.float32)
    for i in range(RPD // 16):
        stage_v[pl.ds(i * 16, 16)] = jnp.zeros((16,), jnp.float32)
    pltpu.sync_copy(stage_v, acc.at[pl.ds(s * RPD, RPD)])
    pltpu.sync_copy(dst_hbm.at[w], dst_v)
    plsc.subcore_barrier()

    def body(j, carry):
        pltpu.sync_copy(ones_v, acc.at[dst_v.at[j]], add=True)
        return carry

    lax.fori_loop(0, KD, body, 0)
    plsc.subcore_barrier()
    pltpu.sync_copy(acc.at[pl.ds(s * RPD, RPD)], stage_v)

    @pl.when(c == 0)
    def _():
        pltpu.sync_copy(stage_v, out0_hbm.at[pl.ds(s * RPD, RPD)])

    @pl.when(c == 1)
    def _():
        pltpu.sync_copy(stage_v, out1_hbm.at[pl.ds(s * RPD, RPD)])


def _deg_call(dst3):
    k = functools.partial(
        pl.kernel,
        out_type=[
            jax.ShapeDtypeStruct((NP,), jnp.float32),
            jax.ShapeDtypeStruct((NP,), jnp.float32),
        ],
        mesh=_sc_mesh(),
        scratch_types=[
            pltpu.VMEM((KD, CD), jnp.int32),
            pltpu.VMEM((CD,), jnp.float32),
            pltpu.VMEM((RPD,), jnp.float32),
            pltpu.VMEM_SHARED((NP,), jnp.float32),
        ],
    )(_deg_body)
    return k(dst3)


def _agg_body(sh_hbm, src_hbm, dst_hbm, out_hbm, sidx, didx, gbuf, acc,
              semi, semg):
    c = lax.axis_index("c")
    s = lax.axis_index("s")
    w = c * NS + s
    r0 = s * RPT
    pltpu.sync_copy(sh_hbm.at[pl.ds(r0, RPT)], acc.at[pl.ds(r0, RPT)])

    @pl.when(s == NS - 1)
    def _():
        pltpu.sync_copy(sh_hbm.at[pl.ds(TAIL0, TAILN)],
                        acc.at[pl.ds(TAIL0, TAILN)])

    plsc.subcore_barrier()

    for i in range(IS):
        pltpu.async_copy(src_hbm.at[w].at[i], sidx.at[i], semi.at[i])
        pltpu.async_copy(dst_hbm.at[w].at[i], didx.at[i], semi.at[i])
    for i in range(NBG):
        pltpu.make_async_copy(src_hbm.at[w].at[i], sidx.at[i],
                              semi.at[i]).wait()
        pltpu.make_async_copy(dst_hbm.at[w].at[i], didx.at[i],
                              semi.at[i]).wait()
        pltpu.async_copy(sh_hbm.at[sidx.at[i]], gbuf.at[i], semg.at[i])

    def body(j0, carry):
        for i in range(IS):
            j = j0 * IS + i
            b = i % NBG

            @pl.when(j < KC)
            def _():
                pltpu.make_async_copy(sh_hbm.at[sidx.at[i]], gbuf.at[b],
                                      semg.at[b]).wait()
                pltpu.sync_copy(gbuf.at[b], acc.at[didx.at[i]], add=True)

            @pl.when(j + IS < KC)
            def _():
                pltpu.async_copy(src_hbm.at[w].at[j + IS], sidx.at[i],
                                 semi.at[i])
                pltpu.async_copy(dst_hbm.at[w].at[j + IS], didx.at[i],
                                 semi.at[i])

            i2 = (i + NBG) % IS

            @pl.when(j + NBG < KC)
            def _():
                pltpu.make_async_copy(src_hbm.at[w].at[j + NBG], sidx.at[i2],
                                      semi.at[i2]).wait()
                pltpu.make_async_copy(dst_hbm.at[w].at[j + NBG], didx.at[i2],
                                      semi.at[i2]).wait()
                pltpu.async_copy(sh_hbm.at[sidx.at[i2]], gbuf.at[b],
                                 semg.at[b])
        return carry

    lax.fori_loop(0, KCE, body, 0)
    plsc.subcore_barrier()
    pltpu.sync_copy(acc.at[pl.ds(r0, RPT)], out_hbm.at[c].at[pl.ds(r0, RPT)])

    @pl.when(s == NS - 1)
    def _():
        pltpu.sync_copy(acc.at[pl.ds(TAIL0, TAILN)],
                        out_hbm.at[c].at[pl.ds(TAIL0, TAILN)])


def _agg_call(sh, src3, dst3):
    k = functools.partial(
        pl.kernel,
        out_type=jax.ShapeDtypeStruct((NC, N_NODES, D), jnp.float32),
        mesh=_sc_mesh(),
        scratch_types=[
            pltpu.VMEM((IS, CH), jnp.int32),
            pltpu.VMEM((IS, CH), jnp.int32),
            pltpu.VMEM((NBG, CH, D), jnp.float32),
            pltpu.VMEM_SHARED((N_NODES, D), jnp.float32),
            pltpu.SemaphoreType.DMA((IS,)),
            pltpu.SemaphoreType.DMA((NBG,)),
        ],
    )(_agg_body)
    return k(sh, src3, dst3)


def _pre1_kernel(degT_ref, x_ref, w1_ref, sh_ref, dinv_ref):
    deg = jnp.sum(degT_ref[...], axis=1, keepdims=True) + 1.0
    dinv = lax.rsqrt(deg)
    h = jnp.dot(x_ref[...], w1_ref[...], preferred_element_type=jnp.float32)
    sh_ref[...] = dinv * h
    dinv_ref[...] = dinv


def _pre1(degT, x, W1):
    return pl.pallas_call(
        _pre1_kernel,
        grid=(GRID,),
        in_specs=[
            pl.BlockSpec((RB, NC), lambda i: (i, 0)),
            pl.BlockSpec((RB, D), lambda i: (i, 0)),
            pl.BlockSpec((D, D), lambda i: (0, 0)),
        ],
        out_specs=[
            pl.BlockSpec((RB, D), lambda i: (i, 0)),
            pl.BlockSpec((RB, 1), lambda i: (i, 0)),
        ],
        out_shape=[
            jax.ShapeDtypeStruct((N_NODES, D), jnp.float32),
            jax.ShapeDtypeStruct((N_NODES, 1), jnp.float32),
        ],
    )(degT, x, W1)


def _comb1_kernel(p_ref, sh_ref, dinv_ref, b1_ref, t_ref, stats_ref):
    agg = p_ref[0] + p_ref[1] - sh_ref[...]
    t = dinv_ref[...] * agg + b1_ref[...]
    t_ref[...] = t

    @pl.when(pl.program_id(0) == 0)
    def _():
        stats_ref[...] = jnp.zeros_like(stats_ref)

    stats_ref[0:1, :] += jnp.sum(t, axis=0, keepdims=True)
    stats_ref[1:2, :] += jnp.sum(t * t, axis=0, keepdims=True)


def _comb1(p01, sh1, dinv, b1):
    return pl.pallas_call(
        _comb1_kernel,
        grid=(GRID,),
        in_specs=[
            pl.BlockSpec((NC, RB, D), lambda i: (0, i, 0)),
            pl.BlockSpec((RB, D), lambda i: (i, 0)),
            pl.BlockSpec((RB, 1), lambda i: (i, 0)),
            pl.BlockSpec((1, D), lambda i: (0, 0)),
        ],
        out_specs=[
            pl.BlockSpec((RB, D), lambda i: (i, 0)),
            pl.BlockSpec((8, D), lambda i: (0, 0)),
        ],
        out_shape=[
            jax.ShapeDtypeStruct((N_NODES, D), jnp.float32),
            jax.ShapeDtypeStruct((8, D), jnp.float32),
        ],
    )(p01, sh1, dinv, b1)


def _bn2_kernel(t_ref, stats_ref, g_ref, be_ref, dinv_ref, w2_ref, sh2_ref):
    n = jnp.float32(N_NODES)
    mean = stats_ref[0:1, :] / n
    var = stats_ref[1:2, :] / n - mean * mean
    inv = lax.rsqrt(var + 1e-5)
    h = (t_ref[...] - mean) * inv * g_ref[...] + be_ref[...]
    h = jnp.maximum(h, 0.0)
    sh2_ref[...] = dinv_ref[...] * jnp.dot(
        h, w2_ref[...], preferred_element_type=jnp.float32)


def _bn2(t, stats, gamma1, beta1, dinv, W2):
    return pl.pallas_call(
        _bn2_kernel,
        grid=(GRID,),
        in_specs=[
            pl.BlockSpec((RB, D), lambda i: (i, 0)),
            pl.BlockSpec((8, D), lambda i: (0, 0)),
            pl.BlockSpec((1, D), lambda i: (0, 0)),
            pl.BlockSpec((1, D), lambda i: (0, 0)),
            pl.BlockSpec((RB, 1), lambda i: (i, 0)),
            pl.BlockSpec((D, D), lambda i: (0, 0)),
        ],
        out_specs=pl.BlockSpec((RB, D), lambda i: (i, 0)),
        out_shape=jax.ShapeDtypeStruct((N_NODES, D), jnp.float32),
    )(t, stats, gamma1, beta1, dinv, W2)


def _head_kernel(q_ref, sh2_ref, dinv_ref, b2_ref, wp1_ref, bp1_ref, wp2_ref,
                 bp2_ref, out_ref):
    agg = q_ref[0] + q_ref[1] - sh2_ref[...]
    h = dinv_ref[...] * agg + b2_ref[...]
    p = jnp.tanh(
        jnp.dot(h, wp1_ref[...], preferred_element_type=jnp.float32)
        + bp1_ref[...])
    out_ref[...] = jnp.dot(
        p, wp2_ref[...], preferred_element_type=jnp.float32) + bp2_ref[...]


def _head(q01, sh2, dinv, b2, Wp1, bp1, Wp2, bp2):
    return pl.pallas_call(
        _head_kernel,
        grid=(GRID,),
        in_specs=[
            pl.BlockSpec((NC, RB, D), lambda i: (0, i, 0)),
            pl.BlockSpec((RB, D), lambda i: (i, 0)),
            pl.BlockSpec((RB, 1), lambda i: (i, 0)),
            pl.BlockSpec((1, D), lambda i: (0, 0)),
            pl.BlockSpec((D, D), lambda i: (0, 0)),
            pl.BlockSpec((1, D), lambda i: (0, 0)),
            pl.BlockSpec((D, D), lambda i: (0, 0)),
            pl.BlockSpec((1, D), lambda i: (0, 0)),
        ],
        out_specs=pl.BlockSpec((RB, D), lambda i: (i, 0)),
        out_shape=jax.ShapeDtypeStruct((N_NODES, D), jnp.float32),
    )(q01, sh2, dinv, b2, Wp1, bp1, Wp2, bp2)


def kernel(x, edge_index, W1, b1, gamma1, beta1, W2, b2, Wp1, bp1, Wp2, bp2):
    ei = edge_index.astype(jnp.int32)
    src3 = ei[0].reshape(NW, KC, CH)
    dst3 = ei[1].reshape(NW, KC, CH)
    dst3d = ei[1].reshape(NW, KD, CD)
    b1r = b1.reshape(1, D)
    b2r = b2.reshape(1, D)
    g1r = gamma1.reshape(1, D)
    be1r = beta1.reshape(1, D)
    bp1r = bp1.reshape(1, D)
    bp2r = bp2.reshape(1, D)

    deg0, deg1 = _deg_call(dst3d)
    degT = jnp.stack([deg0, deg1], axis=1)
    sh1, dinv = _pre1(degT, x, W1)
    p01 = _agg_call(sh1, src3, dst3)
    t, stats = _comb1(p01, sh1, dinv, b1r)
    sh2 = _bn2(t, stats, g1r, be1r, dinv, W2)
    q01 = _agg_call(sh2, src3, dst3)
    return _head(q01, sh2, dinv, b2r, Wp1, bp1r, Wp2, bp2r)


# speedup vs baseline: 1.1717x; 1.0400x over previous
import functools

import jax
import jax.numpy as jnp
from jax import lax
from jax.experimental import pallas as pl
from jax.experimental.pallas import tpu as pltpu
from jax.experimental.pallas import tpu_sc as plsc

N_NODES = 10000
N_EDGES = 320000
D = 128

NC = 2
NS = 16
NW = NC * NS
EPW = N_EDGES // NW
CH = 40
KC = EPW // CH
NBG = 5
IS = 2 * NBG
KCE = -(-KC // IS)
CD = 80
KD = EPW // CD
RPT = 624
TAIL0 = RPT * NS
TAILN = N_NODES - TAIL0
NP = 10240
RPD = NP // NS

RB = 2000
GRID = N_NODES // RB


def _sc_mesh():
    return plsc.VectorSubcoreMesh(core_axis_name="c", subcore_axis_name="s")


def _deg_body(dst_hbm, out0_hbm, out1_hbm, dst_v, ones_v, stage_v, acc):
    c = lax.axis_index("c")
    s = lax.axis_index("s")
    w = c * NS + s
    for i in range(CD // 16):
        ones_v[pl.ds(i * 16, 16)] = jnp.ones((16,), jnp.float32)
    for i in range(RPD // 16):
        stage_v[pl.ds(i * 16, 16)] = jnp.zeros((16,), jnp.float32)
    pltpu.sync_copy(stage_v, acc.at[pl.ds(s * RPD, RPD)])
    pltpu.sync_copy(dst_hbm.at[w], dst_v)
    plsc.subcore_barrier()

    def body(j, carry):
        pltpu.sync_copy(ones_v, acc.at[dst_v.at[j]], add=True)
        return carry

    lax.fori_loop(0, KD, body, 0)
    plsc.subcore_barrier()
    pltpu.sync_copy(acc.at[pl.ds(s * RPD, RPD)], stage_v)

    @pl.when(c == 0)
    def _():
        pltpu.sync_copy(stage_v, out0_hbm.at[pl.ds(s * RPD, RPD)])

    @pl.when(c == 1)
    def _():
        pltpu.sync_copy(stage_v, out1_hbm.at[pl.ds(s * RPD, RPD)])


def _deg_call(dst3):
    k = functools.partial(
        pl.kernel,
        out_type=[
            jax.ShapeDtypeStruct((NP,), jnp.float32),
            jax.ShapeDtypeStruct((NP,), jnp.float32),
        ],
        mesh=_sc_mesh(),
        scratch_types=[
            pltpu.VMEM((KD, CD), jnp.int32),
            pltpu.VMEM((CD,), jnp.float32),
            pltpu.VMEM((RPD,), jnp.float32),
            pltpu.VMEM_SHARED((NP,), jnp.float32),
        ],
    )(_deg_body)
    return k(dst3)


def _agg_body(sh_hbm, src_hbm, dst_hbm, out_hbm, sidx, didx, gbuf, acc,
              semi, semg):
    c = lax.axis_index("c")
    s = lax.axis_index("s")
    w = c * NS + s
    r0 = s * RPT
    pltpu.sync_copy(sh_hbm.at[pl.ds(r0, RPT)], acc.at[pl.ds(r0, RPT)])

    @pl.when(s == NS - 1)
    def _():
        pltpu.sync_copy(sh_hbm.at[pl.ds(TAIL0, TAILN)],
                        acc.at[pl.ds(TAIL0, TAILN)])

    plsc.subcore_barrier()

    for i in range(IS):
        pltpu.async_copy(src_hbm.at[w].at[i], sidx.at[i], semi.at[i])
        pltpu.async_copy(dst_hbm.at[w].at[i], didx.at[i], semi.at[i])
    for i in range(NBG):
        pltpu.make_async_copy(src_hbm.at[w].at[i], sidx.at[i],
                              semi.at[i]).wait()
        pltpu.make_async_copy(dst_hbm.at[w].at[i], didx.at[i],
                              semi.at[i]).wait()
        pltpu.async_copy(sh_hbm.at[sidx.at[i]], gbuf.at[i], semg.at[i])

    def body(j0, carry):
        for i in range(IS):
            j = j0 * IS + i
            b = i % NBG

            @pl.when(j < KC)
            def _():
                pltpu.make_async_copy(sh_hbm.at[sidx.at[i]], gbuf.at[b],
                                      semg.at[b]).wait()
                pltpu.sync_copy(gbuf.at[b], acc.at[didx.at[i]], add=True)

            @pl.when(j + IS < KC)
            def _():
                pltpu.async_copy(src_hbm.at[w].at[j + IS], sidx.at[i],
                                 semi.at[i])
                pltpu.async_copy(dst_hbm.at[w].at[j + IS], didx.at[i],
                                 semi.at[i])

            i2 = (i + NBG) % IS

            @pl.when(j + NBG < KC)
            def _():
                pltpu.make_async_copy(src_hbm.at[w].at[j + NBG], sidx.at[i2],
                                      semi.at[i2]).wait()
                pltpu.make_async_copy(dst_hbm.at[w].at[j + NBG], didx.at[i2],
                                      semi.at[i2]).wait()
                pltpu.async_copy(sh_hbm.at[sidx.at[i2]], gbuf.at[b],
                                 semg.at[b])
        return carry

    lax.fori_loop(0, KCE, body, 0)
    plsc.subcore_barrier()
    pltpu.sync_copy(acc.at[pl.ds(r0, RPT)], out_hbm.at[c].at[pl.ds(r0, RPT)])

    @pl.when(s == NS - 1)
    def _():
        pltpu.sync_copy(acc.at[pl.ds(TAIL0, TAILN)],
                        out_hbm.at[c].at[pl.ds(TAIL0, TAILN)])


def _agg_call(sh, src3, dst3):
    k = functools.partial(
        pl.kernel,
        out_type=jax.ShapeDtypeStruct((NC, N_NODES, D), jnp.float32),
        mesh=_sc_mesh(),
        scratch_types=[
            pltpu.VMEM((IS, CH), jnp.int32),
            pltpu.VMEM((IS, CH), jnp.int32),
            pltpu.VMEM((NBG, CH, D), jnp.float32),
            pltpu.VMEM_SHARED((N_NODES, D), jnp.float32),
            pltpu.SemaphoreType.DMA((IS,)),
            pltpu.SemaphoreType.DMA((NBG,)),
        ],
    )(_agg_body)
    return k(sh, src3, dst3)


def _pre1_kernel(degT_ref, x_ref, w1_ref, sh_ref, dinv_ref):
    deg = jnp.sum(degT_ref[...], axis=1, keepdims=True) + 1.0
    dinv = lax.rsqrt(deg)
    h = jnp.dot(x_ref[...], w1_ref[...], preferred_element_type=jnp.float32)
    sh_ref[...] = dinv * h
    dinv_ref[...] = dinv


def _pre1(degT, x, W1):
    return pl.pallas_call(
        _pre1_kernel,
        grid=(GRID,),
        in_specs=[
            pl.BlockSpec((RB, NC), lambda i: (i, 0)),
            pl.BlockSpec((RB, D), lambda i: (i, 0)),
            pl.BlockSpec((D, D), lambda i: (0, 0)),
        ],
        out_specs=[
            pl.BlockSpec((RB, D), lambda i: (i, 0)),
            pl.BlockSpec((RB, 1), lambda i: (i, 0)),
        ],
        out_shape=[
            jax.ShapeDtypeStruct((N_NODES, D), jnp.float32),
            jax.ShapeDtypeStruct((N_NODES, 1), jnp.float32),
        ],
    )(degT, x, W1)


def _comb1_kernel(p_ref, sh_ref, dinv_ref, b1_ref, t_ref, stats_ref):
    agg = p_ref[0] + p_ref[1] - sh_ref[...]
    t = dinv_ref[...] * agg + b1_ref[...]
    t_ref[...] = t

    @pl.when(pl.program_id(0) == 0)
    def _():
        stats_ref[...] = jnp.zeros_like(stats_ref)

    stats_ref[0:1, :] += jnp.sum(t, axis=0, keepdims=True)
    stats_ref[1:2, :] += jnp.sum(t * t, axis=0, keepdims=True)


def _comb1(p01, sh1, dinv, b1):
    return pl.pallas_call(
        _comb1_kernel,
        grid=(GRID,),
        in_specs=[
            pl.BlockSpec((NC, RB, D), lambda i: (0, i, 0)),
            pl.BlockSpec((RB, D), lambda i: (i, 0)),
            pl.BlockSpec((RB, 1), lambda i: (i, 0)),
            pl.BlockSpec((1, D), lambda i: (0, 0)),
        ],
        out_specs=[
            pl.BlockSpec((RB, D), lambda i: (i, 0)),
            pl.BlockSpec((8, D), lambda i: (0, 0)),
        ],
        out_shape=[
            jax.ShapeDtypeStruct((N_NODES, D), jnp.float32),
            jax.ShapeDtypeStruct((8, D), jnp.float32),
        ],
    )(p01, sh1, dinv, b1)


def _bn2_kernel(t_ref, stats_ref, g_ref, be_ref, dinv_ref, w2_ref, sh2_ref):
    n = jnp.float32(N_NODES)
    mean = stats_ref[0:1, :] / n
    var = stats_ref[1:2, :] / n - mean * mean
    inv = lax.rsqrt(var + 1e-5)
    h = (t_ref[...] - mean) * inv * g_ref[...] + be_ref[...]
    h = jnp.maximum(h, 0.0)
    sh2_ref[...] = dinv_ref[...] * jnp.dot(
        h, w2_ref[...], preferred_element_type=jnp.float32)


def _bn2(t, stats, gamma1, beta1, dinv, W2):
    return pl.pallas_call(
        _bn2_kernel,
        grid=(GRID,),
        in_specs=[
            pl.BlockSpec((RB, D), lambda i: (i, 0)),
            pl.BlockSpec((8, D), lambda i: (0, 0)),
            pl.BlockSpec((1, D), lambda i: (0, 0)),
            pl.BlockSpec((1, D), lambda i: (0, 0)),
            pl.BlockSpec((RB, 1), lambda i: (i, 0)),
            pl.BlockSpec((D, D), lambda i: (0, 0)),
        ],
        out_specs=pl.BlockSpec((RB, D), lambda i: (i, 0)),
        out_shape=jax.ShapeDtypeStruct((N_NODES, D), jnp.float32),
    )(t, stats, gamma1, beta1, dinv, W2)


def _head_kernel(q_ref, sh2_ref, dinv_ref, b2_ref, wp1_ref, bp1_ref, wp2_ref,
                 bp2_ref, out_ref):
    agg = q_ref[0] + q_ref[1] - sh2_ref[...]
    h = dinv_ref[...] * agg + b2_ref[...]
    p = jnp.tanh(
        jnp.dot(h, wp1_ref[...], preferred_element_type=jnp.float32)
        + bp1_ref[...])
    out_ref[...] = jnp.dot(
        p, wp2_ref[...], preferred_element_type=jnp.float32) + bp2_ref[...]


def _head(q01, sh2, dinv, b2, Wp1, bp1, Wp2, bp2):
    return pl.pallas_call(
        _head_kernel,
        grid=(GRID,),
        in_specs=[
            pl.BlockSpec((NC, RB, D), lambda i: (0, i, 0)),
            pl.BlockSpec((RB, D), lambda i: (i, 0)),
            pl.BlockSpec((RB, 1), lambda i: (i, 0)),
            pl.BlockSpec((1, D), lambda i: (0, 0)),
            pl.BlockSpec((D, D), lambda i: (0, 0)),
            pl.BlockSpec((1, D), lambda i: (0, 0)),
            pl.BlockSpec((D, D), lambda i: (0, 0)),
            pl.BlockSpec((1, D), lambda i: (0, 0)),
        ],
        out_specs=pl.BlockSpec((RB, D), lambda i: (i, 0)),
        out_shape=jax.ShapeDtypeStruct((N_NODES, D), jnp.float32),
    )(q01, sh2, dinv, b2, Wp1, bp1, Wp2, bp2)


def kernel(x, edge_index, W1, b1, gamma1, beta1, W2, b2, Wp1, bp1, Wp2, bp2):
    ei = edge_index.astype(jnp.int32)
    src3 = ei[0].reshape(NW, KC, CH)
    dst3 = ei[1].reshape(NW, KC, CH)
    dst3d = ei[1].reshape(NW, KD, CD)
    b1r = b1.reshape(1, D)
    b2r = b2.reshape(1, D)
    g1r = gamma1.reshape(1, D)
    be1r = beta1.reshape(1, D)
    bp1r = bp1.reshape(1, D)
    bp2r = bp2.reshape(1, D)

    deg0, deg1 = _deg_call(dst3d)
    degT = jnp.stack([deg0, deg1], axis=1)
    sh1, dinv = _pre1(degT, x, W1)
    p01 = _agg_call(sh1, src3, dst3)
    t, stats = _comb1(p01, sh1, dinv, b1r)
    sh2 = _bn2(t, stats, g1r, be1r, dinv, W2)
    q01 = _agg_call(sh2, src3, dst3)
    return _head(q01, sh2, dinv, b2r, Wp1, bp1r, Wp2, bp2r)


# speedup vs baseline: 1.1961x; 1.0208x over previous
import functools

import jax
import jax.numpy as jnp
from jax import lax
from jax.experimental import pallas as pl
from jax.experimental.pallas import tpu as pltpu
from jax.experimental.pallas import tpu_sc as plsc

N_NODES = 10000
N_EDGES = 320000
D = 128

NC = 2
NS = 16
NW = NC * NS
EPW = N_EDGES // NW
CH = 40
KC = EPW // CH
NBG = 5
IS = 2 * NBG
KCE = -(-KC // IS)
CD = 80
KD = EPW // CD
RPT = 624
TAIL0 = RPT * NS
TAILN = N_NODES - TAIL0
NP = 10240
RPD = NP // NS

RB = 2000
GRID = N_NODES // RB


def _sc_mesh():
    return plsc.VectorSubcoreMesh(core_axis_name="c", subcore_axis_name="s")


def _deg_body(dst_hbm, out0_hbm, out1_hbm, dst_v, ones_v, stage_v, acc, sem):
    c = lax.axis_index("c")
    s = lax.axis_index("s")
    w = c * NS + s
    for i in range(CD // 16):
        ones_v[pl.ds(i * 16, 16)] = jnp.ones((16,), jnp.float32)
    for i in range(RPD // 16):
        stage_v[pl.ds(i * 16, 16)] = jnp.zeros((16,), jnp.float32)
    pltpu.sync_copy(stage_v, acc.at[pl.ds(s * RPD, RPD)])
    pltpu.sync_copy(dst_hbm.at[w], dst_v)
    plsc.subcore_barrier()

    def body(j, carry):
        pltpu.async_copy(ones_v, acc.at[dst_v.at[j]], sem, add=True)
        return carry

    lax.fori_loop(0, KD, body, 0)

    def drain(j, carry):
        pltpu.make_async_copy(ones_v, acc.at[dst_v.at[0]], sem).wait()
        return carry

    lax.fori_loop(0, KD, drain, 0)
    plsc.subcore_barrier()
    pltpu.sync_copy(acc.at[pl.ds(s * RPD, RPD)], stage_v)

    @pl.when(c == 0)
    def _():
        pltpu.sync_copy(stage_v, out0_hbm.at[pl.ds(s * RPD, RPD)])

    @pl.when(c == 1)
    def _():
        pltpu.sync_copy(stage_v, out1_hbm.at[pl.ds(s * RPD, RPD)])


def _deg_call(dst3):
    k = functools.partial(
        pl.kernel,
        out_type=[
            jax.ShapeDtypeStruct((NP,), jnp.float32),
            jax.ShapeDtypeStruct((NP,), jnp.float32),
        ],
        mesh=_sc_mesh(),
        scratch_types=[
            pltpu.VMEM((KD, CD), jnp.int32),
            pltpu.VMEM((CD,), jnp.float32),
            pltpu.VMEM((RPD,), jnp.float32),
            pltpu.VMEM_SHARED((NP,), jnp.float32),
            pltpu.SemaphoreType.DMA,
        ],
    )(_deg_body)
    return k(dst3)


def _agg_body(sh_hbm, src_hbm, dst_hbm, out_hbm, sidx, didx, gbuf, acc,
              semi, semg):
    c = lax.axis_index("c")
    s = lax.axis_index("s")
    w = c * NS + s
    r0 = s * RPT
    pltpu.sync_copy(sh_hbm.at[pl.ds(r0, RPT)], acc.at[pl.ds(r0, RPT)])

    @pl.when(s == NS - 1)
    def _():
        pltpu.sync_copy(sh_hbm.at[pl.ds(TAIL0, TAILN)],
                        acc.at[pl.ds(TAIL0, TAILN)])

    plsc.subcore_barrier()

    for i in range(IS):
        pltpu.async_copy(src_hbm.at[w].at[i], sidx.at[i], semi.at[i])
        pltpu.async_copy(dst_hbm.at[w].at[i], didx.at[i], semi.at[i])
    for i in range(NBG):
        pltpu.make_async_copy(src_hbm.at[w].at[i], sidx.at[i],
                              semi.at[i]).wait()
        pltpu.make_async_copy(dst_hbm.at[w].at[i], didx.at[i],
                              semi.at[i]).wait()
        pltpu.async_copy(sh_hbm.at[sidx.at[i]], gbuf.at[i], semg.at[i])

    def body(j0, carry):
        for i in range(IS):
            j = j0 * IS + i
            b = i % NBG

            @pl.when(j < KC)
            def _():
                pltpu.make_async_copy(sh_hbm.at[sidx.at[i]], gbuf.at[b],
                                      semg.at[b]).wait()
                pltpu.sync_copy(gbuf.at[b], acc.at[didx.at[i]], add=True)

            @pl.when(j + IS < KC)
            def _():
                pltpu.async_copy(src_hbm.at[w].at[j + IS], sidx.at[i],
                                 semi.at[i])
                pltpu.async_copy(dst_hbm.at[w].at[j + IS], didx.at[i],
                                 semi.at[i])

            i2 = (i + NBG) % IS

            @pl.when(j + NBG < KC)
            def _():
                pltpu.make_async_copy(src_hbm.at[w].at[j + NBG], sidx.at[i2],
                                      semi.at[i2]).wait()
                pltpu.make_async_copy(dst_hbm.at[w].at[j + NBG], didx.at[i2],
                                      semi.at[i2]).wait()
                pltpu.async_copy(sh_hbm.at[sidx.at[i2]], gbuf.at[b],
                                 semg.at[b])
        return carry

    lax.fori_loop(0, KCE, body, 0)
    plsc.subcore_barrier()
    pltpu.sync_copy(acc.at[pl.ds(r0, RPT)], out_hbm.at[c].at[pl.ds(r0, RPT)])

    @pl.when(s == NS - 1)
    def _():
        pltpu.sync_copy(acc.at[pl.ds(TAIL0, TAILN)],
                        out_hbm.at[c].at[pl.ds(TAIL0, TAILN)])


def _agg_call(sh, src3, dst3):
    k = functools.partial(
        pl.kernel,
        out_type=jax.ShapeDtypeStruct((NC, N_NODES, D), jnp.float32),
        mesh=_sc_mesh(),
        scratch_types=[
            pltpu.VMEM((IS, CH), jnp.int32),
            pltpu.VMEM((IS, CH), jnp.int32),
            pltpu.VMEM((NBG, CH, D), jnp.float32),
            pltpu.VMEM_SHARED((N_NODES, D), jnp.float32),
            pltpu.SemaphoreType.DMA((IS,)),
            pltpu.SemaphoreType.DMA((NBG,)),
        ],
    )(_agg_body)
    return k(sh, src3, dst3)


def _pre1_kernel(degT_ref, x_ref, w1_ref, sh_ref, dinv_ref):
    deg = jnp.sum(degT_ref[...], axis=1, keepdims=True) + 1.0
    dinv = lax.rsqrt(deg)
    h = jnp.dot(x_ref[...], w1_ref[...], preferred_element_type=jnp.float32)
    sh_ref[...] = dinv * h
    dinv_ref[...] = dinv


def _pre1(degT, x, W1):
    return pl.pallas_call(
        _pre1_kernel,
        grid=(GRID,),
        in_specs=[
            pl.BlockSpec((RB, NC), lambda i: (i, 0)),
            pl.BlockSpec((RB, D), lambda i: (i, 0)),
            pl.BlockSpec((D, D), lambda i: (0, 0)),
        ],
        out_specs=[
            pl.BlockSpec((RB, D), lambda i: (i, 0)),
            pl.BlockSpec((RB, 1), lambda i: (i, 0)),
        ],
        out_shape=[
            jax.ShapeDtypeStruct((N_NODES, D), jnp.float32),
            jax.ShapeDtypeStruct((N_NODES, 1), jnp.float32),
        ],
    )(degT, x, W1)


def _comb1_kernel(p_ref, sh_ref, dinv_ref, b1_ref, t_ref, stats_ref):
    agg = p_ref[0] + p_ref[1] - sh_ref[...]
    t = dinv_ref[...] * agg + b1_ref[...]
    t_ref[...] = t

    @pl.when(pl.program_id(0) == 0)
    def _():
        stats_ref[...] = jnp.zeros_like(stats_ref)

    stats_ref[0:1, :] += jnp.sum(t, axis=0, keepdims=True)
    stats_ref[1:2, :] += jnp.sum(t * t, axis=0, keepdims=True)


def _comb1(p01, sh1, dinv, b1):
    return pl.pallas_call(
        _comb1_kernel,
        grid=(GRID,),
        in_specs=[
            pl.BlockSpec((NC, RB, D), lambda i: (0, i, 0)),
            pl.BlockSpec((RB, D), lambda i: (i, 0)),
            pl.BlockSpec((RB, 1), lambda i: (i, 0)),
            pl.BlockSpec((1, D), lambda i: (0, 0)),
        ],
        out_specs=[
            pl.BlockSpec((RB, D), lambda i: (i, 0)),
            pl.BlockSpec((8, D), lambda i: (0, 0)),
        ],
        out_shape=[
            jax.ShapeDtypeStruct((N_NODES, D), jnp.float32),
            jax.ShapeDtypeStruct((8, D), jnp.float32),
        ],
    )(p01, sh1, dinv, b1)


def _bn2_kernel(t_ref, stats_ref, g_ref, be_ref, dinv_ref, w2_ref, sh2_ref):
    n = jnp.float32(N_NODES)
    mean = stats_ref[0:1, :] / n
    var = stats_ref[1:2, :] / n - mean * mean
    inv = lax.rsqrt(var + 1e-5)
    h = (t_ref[...] - mean) * inv * g_ref[...] + be_ref[...]
    h = jnp.maximum(h, 0.0)
    sh2_ref[...] = dinv_ref[...] * jnp.dot(
        h, w2_ref[...], preferred_element_type=jnp.float32)


def _bn2(t, stats, gamma1, beta1, dinv, W2):
    return pl.pallas_call(
        _bn2_kernel,
        grid=(GRID,),
        in_specs=[
            pl.BlockSpec((RB, D), lambda i: (i, 0)),
            pl.BlockSpec((8, D), lambda i: (0, 0)),
            pl.BlockSpec((1, D), lambda i: (0, 0)),
            pl.BlockSpec((1, D), lambda i: (0, 0)),
            pl.BlockSpec((RB, 1), lambda i: (i, 0)),
            pl.BlockSpec((D, D), lambda i: (0, 0)),
        ],
        out_specs=pl.BlockSpec((RB, D), lambda i: (i, 0)),
        out_shape=jax.ShapeDtypeStruct((N_NODES, D), jnp.float32),
    )(t, stats, gamma1, beta1, dinv, W2)


def _head_kernel(q_ref, sh2_ref, dinv_ref, b2_ref, wp1_ref, bp1_ref, wp2_ref,
                 bp2_ref, out_ref):
    agg = q_ref[0] + q_ref[1] - sh2_ref[...]
    h = dinv_ref[...] * agg + b2_ref[...]
    p = jnp.tanh(
        jnp.dot(h, wp1_ref[...], preferred_element_type=jnp.float32)
        + bp1_ref[...])
    out_ref[...] = jnp.dot(
        p, wp2_ref[...], preferred_element_type=jnp.float32) + bp2_ref[...]


def _head(q01, sh2, dinv, b2, Wp1, bp1, Wp2, bp2):
    return pl.pallas_call(
        _head_kernel,
        grid=(GRID,),
        in_specs=[
            pl.BlockSpec((NC, RB, D), lambda i: (0, i, 0)),
            pl.BlockSpec((RB, D), lambda i: (i, 0)),
            pl.BlockSpec((RB, 1), lambda i: (i, 0)),
            pl.BlockSpec((1, D), lambda i: (0, 0)),
            pl.BlockSpec((D, D), lambda i: (0, 0)),
            pl.BlockSpec((1, D), lambda i: (0, 0)),
            pl.BlockSpec((D, D), lambda i: (0, 0)),
            pl.BlockSpec((1, D), lambda i: (0, 0)),
        ],
        out_specs=pl.BlockSpec((RB, D), lambda i: (i, 0)),
        out_shape=jax.ShapeDtypeStruct((N_NODES, D), jnp.float32),
    )(q01, sh2, dinv, b2, Wp1, bp1, Wp2, bp2)


def kernel(x, edge_index, W1, b1, gamma1, beta1, W2, b2, Wp1, bp1, Wp2, bp2):
    ei = edge_index.astype(jnp.int32)
    src3 = ei[0].reshape(NW, KC, CH)
    dst3 = ei[1].reshape(NW, KC, CH)
    dst3d = ei[1].reshape(NW, KD, CD)
    b1r = b1.reshape(1, D)
    b2r = b2.reshape(1, D)
    g1r = gamma1.reshape(1, D)
    be1r = beta1.reshape(1, D)
    bp1r = bp1.reshape(1, D)
    bp2r = bp2.reshape(1, D)

    deg0, deg1 = _deg_call(dst3d)
    degT = jnp.stack([deg0, deg1], axis=1)
    sh1, dinv = _pre1(degT, x, W1)
    p01 = _agg_call(sh1, src3, dst3)
    t, stats = _comb1(p01, sh1, dinv, b1r)
    sh2 = _bn2(t, stats, g1r, be1r, dinv, W2)
    q01 = _agg_call(sh2, src3, dst3)
    return _head(q01, sh2, dinv, b2r, Wp1, bp1r, Wp2, bp2r)
